# baseline (speedup 1.0000x reference)
import functools

import jax
import jax.numpy as jnp
from jax import lax
from jax.experimental import pallas as pl
from jax.experimental.pallas import tpu as pltpu
from jax.experimental.pallas import tpu_sc as plsc

_BLOCK_ROWS = 2048
_SC_NUM_CORES = 2
_SC_LANES = 16


def _row0_sc_kernel(x_ref, pe_ref, out_ref, row_v, pe_v, *, batch, seq_len, d_model):
    wid = lax.axis_index("s") * _SC_NUM_CORES + lax.axis_index("c")

    @pl.when(wid < batch)
    def _fix_batch_row():
        pltpu.sync_copy(x_ref.at[wid * seq_len], row_v)
        pltpu.sync_copy(pe_ref.at[0], pe_v)
        for i in range(d_model // _SC_LANES):
            sl = pl.ds(i * _SC_LANES, _SC_LANES)
            row_v[sl] = row_v[sl] + pe_v[sl]
        pltpu.sync_copy(row_v, out_ref.at[wid])


def _pe_add_kernel(x_ref, pe_ref, o_ref):
    pe1 = pe_ref[1, :]
    o_ref[...] = x_ref[...] + pe1[None, None, :]


def _splice_kernel(y_ref, row0_ref, o_ref):
    b = pl.program_id(0)
    del y_ref
    o_ref[...] = row0_ref[b, :][None, None, :]


def kernel(x, pos_embeddings):
    b, s, d = x.shape
    x2 = x.reshape(b * s, d)
    x3 = x.reshape(b * s, 1, d)

    row0 = pl.kernel(
        functools.partial(_row0_sc_kernel, batch=b, seq_len=s, d_model=d),
        out_type=jax.ShapeDtypeStruct((b, d), x.dtype),
        mesh=plsc.VectorSubcoreMesh(core_axis_name="c", subcore_axis_name="s"),
        scratch_types=[
            pltpu.VMEM((d,), jnp.float32),
            pltpu.VMEM((d,), jnp.float32),
        ],
    )(x2, pos_embeddings)

    y = pl.pallas_call(
        _pe_add_kernel,
        grid=(b * s // _BLOCK_ROWS,),
        in_specs=[
            pl.BlockSpec((_BLOCK_ROWS, 1, d), lambda i: (i, 0, 0)),
            pl.BlockSpec((2, d), lambda i: (0, 0)),
        ],
        out_specs=pl.BlockSpec((_BLOCK_ROWS, 1, d), lambda i: (i, 0, 0)),
        out_shape=jax.ShapeDtypeStruct((b * s, 1, d), x.dtype),
    )(x3, pos_embeddings)

    out = pl.pallas_call(
        _splice_kernel,
        grid=(b,),
        in_specs=[
            pl.BlockSpec((1, 1, d), lambda i, seq_len=s: (i * seq_len, 0, 0)),
            pl.BlockSpec((b, d), lambda i: (0, 0)),
        ],
        out_specs=pl.BlockSpec((1, 1, d), lambda i, seq_len=s: (i * seq_len, 0, 0)),
        out_shape=jax.ShapeDtypeStruct((b * s, 1, d), x.dtype),
        input_output_aliases={0: 0},
    )(y, row0)
    return out.reshape(b, s, d)


# speedup vs baseline: 6.2281x; 6.2281x over previous
import functools

import jax
import jax.numpy as jnp
from jax import lax
from jax.experimental import pallas as pl
from jax.experimental.pallas import tpu as pltpu
from jax.experimental.pallas import tpu_sc as plsc

_BLOCK_ROWS = 2048
_SC_NUM_CORES = 2
_SC_LANES = 16


def _row0_sc_kernel(x_ref, pe_ref, out_ref, row_v, pe_v, *, batch, seq_len, d_model):
    wid = lax.axis_index("s") * _SC_NUM_CORES + lax.axis_index("c")

    @pl.when(wid < batch)
    def _fix_batch_row():
        pltpu.sync_copy(x_ref.at[wid * seq_len], row_v)
        pltpu.sync_copy(pe_ref.at[0], pe_v)
        for i in range(d_model // _SC_LANES):
            sl = pl.ds(i * _SC_LANES, _SC_LANES)
            row_v[sl] = row_v[sl] + pe_v[sl]
        pltpu.sync_copy(row_v, out_ref.at[wid])


def _pe_add_kernel(x_ref, pe_ref, o_ref):
    pe1 = pe_ref[1, :]
    o_ref[...] = x_ref[...] + pe1[None, :]


def _splice_kernel(y_ref, row0_ref, o_ref):
    b = pl.program_id(0)
    o_ref[...] = y_ref[...]
    o_ref[0, :] = row0_ref[b, :]


def kernel(x, pos_embeddings):
    b, s, d = x.shape
    x2 = x.reshape(b * s, d)

    row0 = pl.kernel(
        functools.partial(_row0_sc_kernel, batch=b, seq_len=s, d_model=d),
        out_type=jax.ShapeDtypeStruct((b, d), x.dtype),
        mesh=plsc.VectorSubcoreMesh(core_axis_name="c", subcore_axis_name="s"),
        scratch_types=[
            pltpu.VMEM((d,), jnp.float32),
            pltpu.VMEM((d,), jnp.float32),
        ],
    )(x2, pos_embeddings)

    y = pl.pallas_call(
        _pe_add_kernel,
        grid=(b * s // _BLOCK_ROWS,),
        in_specs=[
            pl.BlockSpec((_BLOCK_ROWS, d), lambda i: (i, 0)),
            pl.BlockSpec((2, d), lambda i: (0, 0)),
        ],
        out_specs=pl.BlockSpec((_BLOCK_ROWS, d), lambda i: (i, 0)),
        out_shape=jax.ShapeDtypeStruct((b * s, d), x.dtype),
    )(x2, pos_embeddings)

    out = pl.pallas_call(
        _splice_kernel,
        grid=(b,),
        in_specs=[
            pl.BlockSpec((8, d), lambda i, seq_len=s: (i * seq_len // 8, 0)),
            pl.BlockSpec((b, d), lambda i: (0, 0)),
        ],
        out_specs=pl.BlockSpec((8, d), lambda i, seq_len=s: (i * seq_len // 8, 0)),
        out_shape=jax.ShapeDtypeStruct((b * s, d), x.dtype),
        input_output_aliases={0: 0},
    )(y, row0)
    return out.reshape(b, s, d)


# speedup vs baseline: 6.2303x; 1.0003x over previous
import functools

import jax
import jax.numpy as jnp
from jax import lax
from jax.experimental import pallas as pl
from jax.experimental.pallas import tpu as pltpu
from jax.experimental.pallas import tpu_sc as plsc

_BLOCK_ROWS = 2048
_SC_NUM_CORES = 2
_SC_LANES = 16


def _row0_sc_kernel(
    x_ref, pe_ref, out_ref, row_v, pe_v, sem_x, sem_pe, *, batch, seq_len, d_model
):
    wid = lax.axis_index("s") * _SC_NUM_CORES + lax.axis_index("c")
    nw = _SC_NUM_CORES * 16
    per_row = nw // batch
    seg = d_model // per_row
    row = wid // per_row
    col = (wid % per_row) * seg

    cp_x = pltpu.async_copy(x_ref.at[row * seq_len, pl.ds(col, seg)], row_v, sem_x)
    cp_pe = pltpu.async_copy(pe_ref.at[0, pl.ds(col, seg)], pe_v, sem_pe)
    cp_x.wait()
    cp_pe.wait()
    for i in range(seg // _SC_LANES):
        sl = pl.ds(i * _SC_LANES, _SC_LANES)
        row_v[sl] = row_v[sl] + pe_v[sl]
    pltpu.sync_copy(row_v, out_ref.at[row, pl.ds(col, seg)])


def _pe_add_kernel(x_ref, pe_ref, o_ref):
    pe1 = pe_ref[1, :]
    o_ref[...] = x_ref[...] + pe1[None, :]


def _splice_kernel(y_ref, row0_ref, o_ref):
    b = pl.program_id(0)
    o_ref[...] = y_ref[...]
    o_ref[0, :] = row0_ref[b, :]


def kernel(x, pos_embeddings):
    b, s, d = x.shape
    x2 = x.reshape(b * s, d)

    row0 = pl.kernel(
        functools.partial(_row0_sc_kernel, batch=b, seq_len=s, d_model=d),
        out_type=jax.ShapeDtypeStruct((b, d), x.dtype),
        mesh=plsc.VectorSubcoreMesh(core_axis_name="c", subcore_axis_name="s"),
        scratch_types=[
            pltpu.VMEM((d // (_SC_NUM_CORES * 16 // b),), jnp.float32),
            pltpu.VMEM((d // (_SC_NUM_CORES * 16 // b),), jnp.float32),
            pltpu.SemaphoreType.DMA,
            pltpu.SemaphoreType.DMA,
        ],
    )(x2, pos_embeddings)

    y = pl.pallas_call(
        _pe_add_kernel,
        grid=(b * s // _BLOCK_ROWS,),
        in_specs=[
            pl.BlockSpec((_BLOCK_ROWS, d), lambda i: (i, 0)),
            pl.BlockSpec((2, d), lambda i: (0, 0)),
        ],
        out_specs=pl.BlockSpec((_BLOCK_ROWS, d), lambda i: (i, 0)),
        out_shape=jax.ShapeDtypeStruct((b * s, d), x.dtype),
    )(x2, pos_embeddings)

    out = pl.pallas_call(
        _splice_kernel,
        grid=(b,),
        in_specs=[
            pl.BlockSpec((8, d), lambda i, seq_len=s: (i * seq_len // 8, 0)),
            pl.BlockSpec((b, d), lambda i: (0, 0)),
        ],
        out_specs=pl.BlockSpec((8, d), lambda i, seq_len=s: (i * seq_len // 8, 0)),
        out_shape=jax.ShapeDtypeStruct((b * s, d), x.dtype),
        input_output_aliases={0: 0},
    )(y, row0)
    return out.reshape(b, s, d)
